# baseline (speedup 1.0000x reference)
import functools

import jax
import jax.numpy as jnp
from jax import lax
from jax.experimental import pallas as pl
from jax.experimental.pallas import tpu as pltpu
from jax.experimental.pallas import tpu_sc as plsc

NC = 2
NS = 16
NW = NC * NS
LANES = 16

P = 8
CB = 400
BT = 2048


def _sc_gather(piece, s_piece, b_len):
    rows = s_piece * b_len
    per_w = rows // NW
    n_chunks = per_w // CB
    mesh = plsc.VectorSubcoreMesh(core_axis_name="c", subcore_axis_name="s")

    @functools.partial(
        pl.kernel,
        out_type=jax.ShapeDtypeStruct((rows, 128), jnp.float32),
        mesh=mesh,
        compiler_params=pltpu.CompilerParams(
            use_tc_tiling_on_sc=False, needs_layout_passes=False
        ),
        scratch_types=[
            pltpu.VMEM((2, CB), jnp.int32),
            pltpu.VMEM((CB, 128), jnp.float32),
            pltpu.VMEM((CB, 128), jnp.float32),
            pltpu.SemaphoreType.DMA,
            pltpu.SemaphoreType.DMA,
            pltpu.SemaphoreType.DMA,
        ],
    )
    def gath(idx_hbm, tbl_hbm, g_hbm, idx_v, gb0, gb1, gsem, ws0, ws1):
        wid = lax.axis_index("s") * NC + lax.axis_index("c")
        base = piece * rows + wid * per_w
        gbs = (gb0, gb1)
        wss = (ws0, ws1)
        wb = [None, None]
        for k in range(n_chunks):
            par = k % 2
            gb = gbs[par]
            if wb[par] is not None:
                wb[par].wait()
            off = k * CB
            pltpu.sync_copy(idx_hbm.at[pl.ds(base + off, CB)], idx_v.at[par])
            pltpu.async_copy(tbl_hbm.at[idx_v.at[par]], gb, gsem).wait()
            wb[par] = pltpu.async_copy(
                gb, g_hbm.at[pl.ds(wid * per_w + off, CB)], wss[par]
            )
        for w in wb:
            if w is not None:
                w.wait()

    return gath


def _tc_add(piece, s_piece, s_len, d, b_len, aliased):
    nb = b_len // BT

    def body(*refs):
        if aliased:
            _, g_ref, x_ref, o_ref = refs
        else:
            g_ref, x_ref, o_ref = refs
        t = jnp.transpose(g_ref[...])
        o_ref[...] = x_ref[...] + t[:d, :]

    in_specs = [
        pl.BlockSpec((BT, 128), lambda i, j: (i * nb + j, 0)),
        pl.BlockSpec((None, d, BT), lambda i, j: (piece * s_piece + i, 0, j)),
    ]
    kwargs = {}
    if aliased:
        in_specs = [pl.BlockSpec(memory_space=pl.ANY)] + in_specs
        kwargs["input_output_aliases"] = {0: 0}
    return pl.pallas_call(
        body,
        grid=(s_piece, nb),
        in_specs=in_specs,
        out_specs=pl.BlockSpec((None, d, BT), lambda i, j: (piece * s_piece + i, 0, j)),
        out_shape=jax.ShapeDtypeStruct((s_len, d, b_len), jnp.float32),
        **kwargs,
    )


@jax.jit
def _embed_add(xt, idx_lin, tbl2):
    s_len, d, b_len = xt.shape
    s_piece = s_len // P
    gs = [_sc_gather(p, s_piece, b_len)(idx_lin, tbl2) for p in range(P)]
    out = _tc_add(0, s_piece, s_len, d, b_len, False)(gs[0], xt)
    for p in range(1, P):
        out = _tc_add(p, s_piece, s_len, d, b_len, True)(out, gs[p], xt)
    return out


def kernel(x, in_chan_matrix, embed_weight):
    b, l, d = x.shape
    v = embed_weight.shape[0]
    xt = jnp.transpose(x, (1, 2, 0))
    idx_lin = in_chan_matrix.astype(jnp.int32).T.reshape(b * l)
    tblp = jnp.pad(embed_weight, ((0, 0), (0, d)))
    ot = _embed_add(xt, idx_lin, tblp)
    return jnp.transpose(ot, (2, 0, 1))
